# baseline (speedup 1.0000x reference)
import jax
import jax.numpy as jnp
from jax.experimental import pallas as pl
from jax.experimental.pallas import tpu as pltpu

B = 4096
F = 16384
BR = 64
NB = B // BR


def _body(feat_ref, w_ref, out_ref, acc_ref):
    i = pl.program_id(0)
    part = jnp.sum(feat_ref[...] * w_ref[...], axis=1)
    padded = jnp.concatenate(
        [part, jnp.full((128 - BR,), -jnp.inf, jnp.float32)])
    acc_ref[pl.ds(i, 1), :] = padded[None, :]

    @pl.when(i == pl.num_programs(0) - 1)
    def _():
        s = acc_ref[...]
        m = jnp.max(s)
        e = jnp.exp(s - m)
        out_ref[...] = e / jnp.sum(e)


def kernel(feature_vector, linear):
    w_row = linear.reshape(1, F)
    probs = pl.pallas_call(
        _body,
        grid=(NB,),
        in_specs=[
            pl.BlockSpec((BR, F), lambda i: (i, 0)),
            pl.BlockSpec((1, F), lambda i: (0, 0)),
        ],
        out_specs=pl.BlockSpec((NB, 128), lambda i: (0, 0)),
        out_shape=jax.ShapeDtypeStruct((NB, 128), jnp.float32),
        scratch_shapes=[pltpu.VMEM((NB, 128), jnp.float32)],
    )(feature_vector, w_row)
    return probs[:, :BR].reshape(1, B, 1)


# speedup vs baseline: 1.1774x; 1.1774x over previous
import jax
import jax.numpy as jnp
from jax import lax
from jax.experimental import pallas as pl
from jax.experimental.pallas import tpu as pltpu

B = 4096
F = 16384
BR = 128
NB = B // BR
DEPTH = 3


def _body(w_ref, feat_hbm, out_ref, acc_ref, bufs_ref, sems):
    i = pl.program_id(0)

    @pl.when(i == 0)
    def _():
        for k in range(DEPTH):
            pltpu.make_async_copy(
                feat_hbm.at[pl.ds(k * BR, BR)],
                bufs_ref.at[k],
                sems.at[k],
            ).start()

    slot = lax.rem(i, DEPTH)
    pltpu.make_async_copy(
        feat_hbm.at[pl.ds(i * BR, BR)],
        bufs_ref.at[slot],
        sems.at[slot],
    ).wait()
    part = jnp.sum(bufs_ref[slot] * w_ref[...], axis=1)
    acc_ref[0, pl.ds(i * BR, BR)] = part

    @pl.when(i + DEPTH < NB)
    def _():
        pltpu.make_async_copy(
            feat_hbm.at[pl.ds((i + DEPTH) * BR, BR)],
            bufs_ref.at[slot],
            sems.at[slot],
        ).start()

    @pl.when(i == NB - 1)
    def _():
        s = acc_ref[...]
        m = jnp.max(s)
        e = jnp.exp(s - m)
        out_ref[...] = e / jnp.sum(e)


def kernel(feature_vector, linear):
    w_row = linear.reshape(1, F)
    probs = pl.pallas_call(
        _body,
        grid=(NB,),
        in_specs=[
            pl.BlockSpec((1, F), lambda i: (0, 0)),
            pl.BlockSpec(memory_space=pl.ANY),
        ],
        out_specs=pl.BlockSpec((1, B), lambda i: (0, 0)),
        out_shape=jax.ShapeDtypeStruct((1, B), jnp.float32),
        scratch_shapes=[
            pltpu.VMEM((1, B), jnp.float32),
            pltpu.VMEM((DEPTH, BR, F), jnp.float32),
            pltpu.SemaphoreType.DMA((DEPTH,)),
        ],
    )(w_row, feature_vector)
    return probs.reshape(1, B, 1)
